# baseline (speedup 1.0000x reference)
import functools

import jax
import jax.numpy as jnp
from jax import lax
from jax.experimental import pallas as pl
from jax.experimental.pallas import tpu as pltpu
from jax.experimental.pallas import tpu_sc as plsc

B, L, EMB = 1024, 200, 16
NW = 32
SEQ_PER_W = B // NW
HALF = L // 2
SCALE = 0.5 ** 0.5


def _body(tok_hbm, word_hbm, pos_hbm, out_hbm, idx_v, rows_v, pos_v, sem):
    nc = 2
    wid = lax.axis_index("s") * nc + lax.axis_index("c")

    pltpu.sync_copy(tok_hbm.at[wid], idx_v)
    pltpu.sync_copy(pos_hbm, pos_v)

    c = jnp.full((EMB,), SCALE, dtype=jnp.float32)

    def scale_pos(l, carry):
        pos_v[l] = pos_v[l] * c
        return carry

    lax.fori_loop(0, L, scale_pos, 0, unroll=4)

    def do_seq(s, carry):
        g0 = pltpu.async_copy(word_hbm.at[idx_v.at[s, 0]], rows_v.at[0], sem)
        g1 = pltpu.async_copy(word_hbm.at[idx_v.at[s, 1]], rows_v.at[1], sem)
        g0.wait()
        g1.wait()

        def add_pos(i, carry):
            rows_v[0, i] = rows_v[0, i] * c + pos_v[i]
            rows_v[1, i] = rows_v[1, i] * c + pos_v[HALF + i]
            return carry

        lax.fori_loop(0, HALF, add_pos, 0, unroll=4)

        pltpu.sync_copy(rows_v, out_hbm.at[wid, s])
        return carry

    lax.fori_loop(0, SEQ_PER_W, do_seq, 0)


@jax.jit
def _embed(tok_r, word_table, pos_table):
    mesh = plsc.VectorSubcoreMesh(core_axis_name="c", subcore_axis_name="s")
    f = pl.kernel(
        _body,
        out_type=jax.ShapeDtypeStruct((NW, SEQ_PER_W, 2, HALF, EMB), jnp.float32),
        mesh=mesh,
        scratch_types=[
            pltpu.VMEM((SEQ_PER_W, 2, HALF), jnp.int32),
            pltpu.VMEM((2, HALF, EMB), jnp.float32),
            pltpu.VMEM((L, EMB), jnp.float32),
            pltpu.SemaphoreType.DMA,
        ],
        compiler_params=pltpu.CompilerParams(use_tc_tiling_on_sc=False),
    )
    return f(tok_r, word_table, pos_table)


def kernel(tok_ids, word_table, pos_table):
    tok_r = tok_ids.reshape(NW, SEQ_PER_W, 2, HALF).astype(jnp.int32)
    out = _embed(tok_r, word_table, pos_table)
    return out.reshape(B, L, EMB)


# speedup vs baseline: 1.0350x; 1.0350x over previous
import functools

import jax
import jax.numpy as jnp
from jax import lax
from jax.experimental import pallas as pl
from jax.experimental.pallas import tpu as pltpu
from jax.experimental.pallas import tpu_sc as plsc

B, L, EMB = 1024, 200, 16
NW = 32
SEQ_PER_W = B // NW
HALF = L // 2
SCALE = 0.5 ** 0.5


def _body(tok_hbm, word_hbm, pos_hbm, out_hbm, idx_v, rows_v, pos_v, sem):
    nc = 2
    wid = lax.axis_index("s") * nc + lax.axis_index("c")

    pltpu.sync_copy(tok_hbm.at[wid], idx_v)
    pltpu.sync_copy(pos_hbm, pos_v)

    c = jnp.full((EMB,), SCALE, dtype=jnp.float32)

    def scale_pos(l, carry):
        pos_v[l] = pos_v[l] * c
        return carry

    lax.fori_loop(0, L, scale_pos, 0, unroll=4)

    def fire(s, carry):
        pltpu.make_async_copy(
            word_hbm.at[idx_v.at[s, 0]], rows_v.at[s, 0], sem).start()
        pltpu.make_async_copy(
            word_hbm.at[idx_v.at[s, 1]], rows_v.at[s, 1], sem).start()
        return carry

    lax.fori_loop(0, SEQ_PER_W, fire, 0)

    def drain(s, carry):
        pltpu.make_async_copy(
            word_hbm.at[idx_v.at[0, 0]], rows_v.at[0, 0], sem).wait()
        return carry

    lax.fori_loop(0, 2 * SEQ_PER_W, drain, 0)

    def add_pos_seq(s, carry):
        def add_pos(i, carry2):
            rows_v[s, 0, i] = rows_v[s, 0, i] * c + pos_v[i]
            rows_v[s, 1, i] = rows_v[s, 1, i] * c + pos_v[HALF + i]
            return carry2

        return lax.fori_loop(0, HALF, add_pos, carry, unroll=8)

    lax.fori_loop(0, SEQ_PER_W, add_pos_seq, 0)

    pltpu.sync_copy(rows_v, out_hbm.at[wid])


@jax.jit
def _embed(tok_r, word_table, pos_table):
    mesh = plsc.VectorSubcoreMesh(core_axis_name="c", subcore_axis_name="s")
    f = pl.kernel(
        _body,
        out_type=jax.ShapeDtypeStruct((NW, SEQ_PER_W, 2, HALF, EMB), jnp.float32),
        mesh=mesh,
        scratch_types=[
            pltpu.VMEM((SEQ_PER_W, 2, HALF), jnp.int32),
            pltpu.VMEM((SEQ_PER_W, 2, HALF, EMB), jnp.float32),
            pltpu.VMEM((L, EMB), jnp.float32),
            pltpu.SemaphoreType.DMA,
        ],
        compiler_params=pltpu.CompilerParams(use_tc_tiling_on_sc=False),
    )
    return f(tok_r, word_table, pos_table)


def kernel(tok_ids, word_table, pos_table):
    tok_r = tok_ids.reshape(NW, SEQ_PER_W, 2, HALF).astype(jnp.int32)
    out = _embed(tok_r, word_table, pos_table)
    return out.reshape(B, L, EMB)


# speedup vs baseline: 1.0758x; 1.0394x over previous
import jax
import jax.numpy as jnp
from jax import lax
from jax.experimental import pallas as pl
from jax.experimental.pallas import tpu as pltpu
from jax.experimental.pallas import tpu_sc as plsc

B, L, EMB = 1024, 200, 16
NW = 32
SEQ_PER_W = B // NW
H0, H1 = 104, 96
SCALE = 0.5 ** 0.5


def _body(tok_hbm, word_hbm, pos_hbm, out_hbm, idx_v, rows_v, pos_v, sem):
    nc = 2
    wid = lax.axis_index("s") * nc + lax.axis_index("c")
    b0 = wid * SEQ_PER_W

    pltpu.sync_copy(tok_hbm.at[pl.ds(b0, SEQ_PER_W)], idx_v)
    pltpu.sync_copy(pos_hbm, pos_v)

    c = jnp.full((EMB,), SCALE, dtype=jnp.float32)

    def scale_pos(l, carry):
        pos_v[l] = pos_v[l] * c
        return carry

    lax.fori_loop(0, L, scale_pos, 0, unroll=4)

    def fire(s, carry):
        pltpu.make_async_copy(
            word_hbm.at[idx_v.at[s, pl.ds(0, H0)]],
            rows_v.at[s, pl.ds(0, H0)], sem).start()
        pltpu.make_async_copy(
            word_hbm.at[idx_v.at[s, pl.ds(H0, H1)]],
            rows_v.at[s, pl.ds(H0, H1)], sem).start()
        return carry

    lax.fori_loop(0, SEQ_PER_W, fire, 0)

    def drain(s, carry):
        pltpu.make_async_copy(
            word_hbm.at[idx_v.at[0, pl.ds(0, H0)]],
            rows_v.at[0, pl.ds(0, H0)], sem).wait()
        pltpu.make_async_copy(
            word_hbm.at[idx_v.at[0, pl.ds(H0, H1)]],
            rows_v.at[0, pl.ds(H0, H1)], sem).wait()
        return carry

    lax.fori_loop(0, SEQ_PER_W, drain, 0)

    def add_pos_seq(s, carry):
        def add_pos(i, carry2):
            rows_v[s, i] = rows_v[s, i] * c + pos_v[i]
            return carry2

        return lax.fori_loop(0, L, add_pos, carry, unroll=8)

    lax.fori_loop(0, SEQ_PER_W, add_pos_seq, 0)

    pltpu.sync_copy(rows_v, out_hbm.at[pl.ds(b0, SEQ_PER_W)])


@jax.jit
def _embed(tok_ids, word_table, pos_table):
    mesh = plsc.VectorSubcoreMesh(core_axis_name="c", subcore_axis_name="s")
    f = pl.kernel(
        _body,
        out_type=jax.ShapeDtypeStruct((B, L, EMB), jnp.float32),
        mesh=mesh,
        scratch_types=[
            pltpu.VMEM((SEQ_PER_W, L), jnp.int32),
            pltpu.VMEM((SEQ_PER_W, L, EMB), jnp.float32),
            pltpu.VMEM((L, EMB), jnp.float32),
            pltpu.SemaphoreType.DMA,
        ],
        compiler_params=pltpu.CompilerParams(use_tc_tiling_on_sc=False),
    )
    return f(tok_ids, word_table, pos_table)


def kernel(tok_ids, word_table, pos_table):
    return _embed(tok_ids, word_table, pos_table)
